# baseline (speedup 1.0000x reference)
import jax
import jax.numpy as jnp
from jax.experimental import pallas as pl
from jax.experimental.pallas import tpu as pltpu

K = 5
S1 = 32
ACC1 = 904
IN1 = 1040
SC = 16
P2 = 224
ACC2 = 160
B = 16
G = 4
CL = 48
LANES = 128
M1 = (G - 1) * IN1 + ACC1
M2 = (G - 1) * P2 + ACC2
A1R = G * P2 + SC


def _convs_kernel(x_ref, w1p_ref, w14_ref, w2a_ref, w2b_ref, b1_ref, b2a_ref,
                  b2b_ref, wf1_ref, bf1_ref, wf2_ref, bf2_ref, wf3_ref, bf3_ref,
                  o_ref, xsp, xs4, acc1, a1c, a1cs, acc2a, acc2b, ot, xf):
    for g in range(G):
        xt = jnp.transpose(x_ref[g * 48:(g + 1) * 48, :], (1, 0)).astype(jnp.bfloat16)
        for p in range(2):
            r0 = p * G * IN1 + g * IN1
            for half48, j in ((slice(0, CL), 2 * p), (slice(CL, 2 * CL), 2 * p + 1)):
                xsp[r0:r0 + 1024 - j, half48] = xt[j:1024, :]
                xsp[r0 + 1024 - j:r0 + IN1, half48] = jnp.zeros(
                    (IN1 - 1024 + j, CL), jnp.bfloat16)
        r0 = g * IN1
        xs4[r0:r0 + 1020, :] = xt[4:1024, :]
        xs4[r0 + 1020:r0 + IN1, :] = jnp.zeros((IN1 - 1020, CL), jnp.bfloat16)

    first = True
    for i in range(K):
        for p in range(2):
            s = p * G * IN1 + i * S1
            q = jnp.dot(xsp[s:s + M1, :], w1p_ref[(i * 2 + p) * 96:(i * 2 + p + 1) * 96, :],
                        preferred_element_type=jnp.float32)
            if first:
                acc1[...] = q
                first = False
            else:
                acc1[...] += q
        s = i * S1
        acc1[...] += jnp.dot(xs4[s:s + M1, :], w14_ref[i * CL:(i + 1) * CL, :],
                             preferred_element_type=jnp.float32)

    for g in range(G):
        for ph in range(14):
            r = g * IN1 + 64 * ph
            v = jnp.maximum(
                jnp.maximum(acc1[r:r + 32:2, :], acc1[r + 1:r + 32:2, :]),
                jnp.maximum(acc1[r + 32:r + 64:2, :], acc1[r + 33:r + 64:2, :]))
            q = g * P2 + SC * ph
            a1c[q:q + SC, :] = jnp.maximum(v + b1_ref[...], 0.0).astype(jnp.bfloat16)

    a1c[G * P2:A1R, :] = jnp.zeros((A1R - G * P2, LANES), jnp.bfloat16)

    for j in range(1, K):
        a1cs[(j - 1) * G * P2:j * G * P2, :] = a1c[j:j + G * P2, :]

    for half, (w2_ref, acc2) in enumerate(((w2a_ref, acc2a), (w2b_ref, acc2b))):
        for idx in range(K * K):
            i, j = idx // K, idx % K
            lhs = (a1c[i * SC:i * SC + M2, :] if j == 0 else
                   a1cs[(j - 1) * G * P2 + i * SC:(j - 1) * G * P2 + i * SC + M2, :])
            p = jnp.dot(lhs, w2_ref[idx * LANES:(idx + 1) * LANES, :],
                        preferred_element_type=jnp.float32)
            if idx == 0:
                acc2[...] = p
            else:
                acc2[...] += p

    for g in range(G):
        rows = []
        for acc2, b2_ref in ((acc2a, b2a_ref), (acc2b, b2b_ref)):
            for ph in range(5):
                r = g * P2 + 32 * ph
                v = jnp.maximum(
                    jnp.maximum(acc2[r:r + 16:2, :], acc2[r + 1:r + 16:2, :]),
                    jnp.maximum(acc2[r + 16:r + 32:2, :], acc2[r + 17:r + 32:2, :]))
                rows.append(jnp.maximum(v + b2_ref[...], 0.0))
        slab = jnp.concatenate(rows, axis=0)
        ot[g * LANES:(g + 1) * LANES, :] = jnp.transpose(slab, (1, 0))
    for c in range(8):
        xf[:, c * 80:(c + 1) * 80] = ot[c:G * LANES:8, :].astype(jnp.bfloat16)

    h1 = jnp.maximum(
        jnp.dot(xf[...], wf1_ref[...], preferred_element_type=jnp.float32)
        + bf1_ref[...], 0.0).astype(jnp.bfloat16)
    h2 = jnp.maximum(
        jnp.dot(h1, wf2_ref[...], preferred_element_type=jnp.float32)
        + bf2_ref[...], 0.0).astype(jnp.bfloat16)
    o_ref[...] = (jnp.dot(h2, wf3_ref[...], preferred_element_type=jnp.float32)
                  + bf3_ref[...])


def _block_diag(w, rows):
    eye = jnp.eye(B, dtype=w.dtype)
    return jnp.einsum('ab,tij->taibj', eye, w).reshape(K * K * B * rows, LANES)


@jax.jit
def kernel(x, w1, b1, w2, b2, wf1, bf1, wf2, bf2, wf3, bf3):
    n = x.shape[0]
    nbg = n // (B * G)

    y = x.reshape(nbg * G * B * 3, 1024)

    w1bd = _block_diag(w1.reshape(K * K, 8, LANES)[:, :3, :8], 3).astype(jnp.bfloat16)
    w1t = w1bd.reshape(K, K, CL, LANES)
    w1p = w1t[:, :4].reshape(K, 2, 2 * CL, LANES).reshape(10 * 96, LANES)
    w14 = w1t[:, 4].reshape(K * CL, LANES)
    w2s = w2.reshape(K * K, LANES, LANES)[:, :8, :16]
    w2a = _block_diag(w2s[:, :, :8], 8).astype(jnp.bfloat16)
    w2b = _block_diag(w2s[:, :, 8:], 8).astype(jnp.bfloat16)
    b1p = jnp.tile(b1[:, :8], (1, B))
    b2a = jnp.tile(b2[:, :8], (1, B))
    b2b = jnp.tile(b2[:, 8:16], (1, B))

    wf1r = wf1.reshape(K, K, LANES, LANES)[:, :, :16, :]
    wf1r = wf1r.reshape(K, K, 2, 8, LANES).transpose(3, 2, 0, 1, 4)
    wf1r = jnp.pad(wf1r, ((0, 0), (0, 0), (0, 0), (0, 3), (0, 0)))
    wf1r = wf1r.reshape(640, LANES).astype(jnp.bfloat16)
    wf2b = wf2.astype(jnp.bfloat16)
    wf3b = wf3.astype(jnp.bfloat16)

    out = pl.pallas_call(
        _convs_kernel,
        out_shape=jax.ShapeDtypeStruct((n, LANES), jnp.float32),
        grid_spec=pltpu.PrefetchScalarGridSpec(
            num_scalar_prefetch=0,
            grid=(nbg,),
            in_specs=[
                pl.BlockSpec((G * B * 3, 1024), lambda b: (b, 0)),
                pl.BlockSpec((10 * 96, LANES), lambda b: (0, 0)),
                pl.BlockSpec((K * CL, LANES), lambda b: (0, 0)),
                pl.BlockSpec((K * K * LANES, LANES), lambda b: (0, 0)),
                pl.BlockSpec((K * K * LANES, LANES), lambda b: (0, 0)),
                pl.BlockSpec((1, LANES), lambda b: (0, 0)),
                pl.BlockSpec((1, LANES), lambda b: (0, 0)),
                pl.BlockSpec((1, LANES), lambda b: (0, 0)),
                pl.BlockSpec((640, LANES), lambda b: (0, 0)),
                pl.BlockSpec((1, LANES), lambda b: (0, 0)),
                pl.BlockSpec((LANES, LANES), lambda b: (0, 0)),
                pl.BlockSpec((1, LANES), lambda b: (0, 0)),
                pl.BlockSpec((LANES, LANES), lambda b: (0, 0)),
                pl.BlockSpec((1, LANES), lambda b: (0, 0)),
            ],
            out_specs=pl.BlockSpec((G * B, LANES), lambda b: (b, 0)),
            scratch_shapes=[
                pltpu.VMEM((2 * G * IN1, 2 * CL), jnp.bfloat16),
                pltpu.VMEM((G * IN1, CL), jnp.bfloat16),
                pltpu.VMEM((M1, LANES), jnp.float32),
                pltpu.VMEM((A1R, LANES), jnp.bfloat16),
                pltpu.VMEM((4 * G * P2, LANES), jnp.bfloat16),
                pltpu.VMEM((M2, LANES), jnp.float32),
                pltpu.VMEM((M2, LANES), jnp.float32),
                pltpu.VMEM((G * LANES, 80), jnp.float32),
                pltpu.VMEM((G * B, 640), jnp.bfloat16),
            ],
        ),
        compiler_params=pltpu.CompilerParams(dimension_semantics=("parallel",)),
    )(y, w1p, w14, w2a, w2b, b1p, b2a, b2b, wf1r, bf1, wf2b, bf2, wf3b, bf3)

    return out[:, :10]


# speedup vs baseline: 1.0161x; 1.0161x over previous
import functools

import jax
import jax.numpy as jnp
from jax.experimental import pallas as pl
from jax.experimental.pallas import tpu as pltpu

K = 5
S1 = 32
ACC1 = 904
IN1 = 1040
SC = 16
P2 = 224
ACC2 = 160
B = 16
G = 4
CL = 48
LANES = 128
M1 = (G - 1) * IN1 + ACC1
M2 = (G - 1) * P2 + ACC2
A1R = G * P2 + SC
FC_TILE = 512


def _convs_kernel(x_ref, w1p_ref, w14_ref, w2a_ref, w2b_ref, b1_ref, b2a_ref,
                  b2b_ref, o_ref, xsp, xs4, acc1, a1c, a1cs, acc2a, acc2b):
    for g in range(G):
        xt = jnp.transpose(x_ref[g * 48:(g + 1) * 48, :], (1, 0)).astype(jnp.bfloat16)
        for p in range(2):
            r0 = p * G * IN1 + g * IN1
            for half48, j in ((slice(0, CL), 2 * p), (slice(CL, 2 * CL), 2 * p + 1)):
                xsp[r0:r0 + 1024 - j, half48] = xt[j:1024, :]
                xsp[r0 + 1024 - j:r0 + IN1, half48] = jnp.zeros(
                    (IN1 - 1024 + j, CL), jnp.bfloat16)
        r0 = g * IN1
        xs4[r0:r0 + 1020, :] = xt[4:1024, :]
        xs4[r0 + 1020:r0 + IN1, :] = jnp.zeros((IN1 - 1020, CL), jnp.bfloat16)

    first = True
    for i in range(K):
        for p in range(2):
            s = p * G * IN1 + i * S1
            q = jnp.dot(xsp[s:s + M1, :], w1p_ref[(i * 2 + p) * 96:(i * 2 + p + 1) * 96, :],
                        preferred_element_type=jnp.float32)
            if first:
                acc1[...] = q
                first = False
            else:
                acc1[...] += q
        s = i * S1
        acc1[...] += jnp.dot(xs4[s:s + M1, :], w14_ref[i * CL:(i + 1) * CL, :],
                             preferred_element_type=jnp.float32)

    for g in range(G):
        for ph in range(14):
            r = g * IN1 + 64 * ph
            v = jnp.maximum(
                jnp.maximum(acc1[r:r + 32:2, :], acc1[r + 1:r + 32:2, :]),
                jnp.maximum(acc1[r + 32:r + 64:2, :], acc1[r + 33:r + 64:2, :]))
            q = g * P2 + SC * ph
            a1c[q:q + SC, :] = jnp.maximum(v + b1_ref[...], 0.0).astype(jnp.bfloat16)

    a1c[G * P2:A1R, :] = jnp.zeros((A1R - G * P2, LANES), jnp.bfloat16)

    for j in range(1, K):
        a1cs[(j - 1) * G * P2:j * G * P2, :] = a1c[j:j + G * P2, :]

    for half, (w2_ref, acc2) in enumerate(((w2a_ref, acc2a), (w2b_ref, acc2b))):
        for idx in range(K * K):
            i, j = idx // K, idx % K
            lhs = (a1c[i * SC:i * SC + M2, :] if j == 0 else
                   a1cs[(j - 1) * G * P2 + i * SC:(j - 1) * G * P2 + i * SC + M2, :])
            p = jnp.dot(lhs, w2_ref[idx * LANES:(idx + 1) * LANES, :],
                        preferred_element_type=jnp.float32)
            if idx == 0:
                acc2[...] = p
            else:
                acc2[...] += p

    for g in range(G):
        rows = []
        for acc2, b2_ref in ((acc2a, b2a_ref), (acc2b, b2b_ref)):
            for ph in range(5):
                r = g * P2 + 32 * ph
                v = jnp.maximum(
                    jnp.maximum(acc2[r:r + 16:2, :], acc2[r + 1:r + 16:2, :]),
                    jnp.maximum(acc2[r + 16:r + 32:2, :], acc2[r + 17:r + 32:2, :]))
                rows.append(jnp.maximum(v + b2_ref[...], 0.0))
        slab = jnp.concatenate(rows, axis=0)
        o_ref[g * LANES:(g + 1) * LANES, :] = jnp.transpose(slab, (1, 0))


def _fc_kernel(tile, f_ref, wf1_ref, bf1_ref, wf2_ref, bf2_ref, wf3_ref,
               bf3_ref, o_ref, xsc):
    for c in range(8):
        xsc[:, c * 80:(c + 1) * 80] = f_ref[c:8 * tile:8, :].astype(jnp.bfloat16)
    h1 = jnp.maximum(
        jnp.dot(xsc[...], wf1_ref[...], preferred_element_type=jnp.float32)
        + bf1_ref[...], 0.0).astype(jnp.bfloat16)
    h2 = jnp.maximum(
        jnp.dot(h1, wf2_ref[...], preferred_element_type=jnp.float32)
        + bf2_ref[...], 0.0).astype(jnp.bfloat16)
    o_ref[...] = (jnp.dot(h2, wf3_ref[...], preferred_element_type=jnp.float32)
                  + bf3_ref[...])


def _block_diag(w, rows):
    eye = jnp.eye(B, dtype=w.dtype)
    return jnp.einsum('ab,tij->taibj', eye, w).reshape(K * K * B * rows, LANES)


@jax.jit
def kernel(x, w1, b1, w2, b2, wf1, bf1, wf2, bf2, wf3, bf3):
    n = x.shape[0]
    nbg = n // (B * G)

    y = x.reshape(nbg * G * B * 3, 1024)

    w1bd = _block_diag(w1.reshape(K * K, 8, LANES)[:, :3, :8], 3).astype(jnp.bfloat16)
    w1t = w1bd.reshape(K, K, CL, LANES)
    w1p = w1t[:, :4].reshape(K, 2, 2 * CL, LANES).reshape(10 * 96, LANES)
    w14 = w1t[:, 4].reshape(K * CL, LANES)
    w2s = w2.reshape(K * K, LANES, LANES)[:, :8, :16]
    w2a = _block_diag(w2s[:, :, :8], 8).astype(jnp.bfloat16)
    w2b = _block_diag(w2s[:, :, 8:], 8).astype(jnp.bfloat16)
    b1p = jnp.tile(b1[:, :8], (1, B))
    b2a = jnp.tile(b2[:, :8], (1, B))
    b2b = jnp.tile(b2[:, 8:16], (1, B))

    feats = pl.pallas_call(
        _convs_kernel,
        out_shape=jax.ShapeDtypeStruct((nbg * G * LANES, 80), jnp.float32),
        grid_spec=pltpu.PrefetchScalarGridSpec(
            num_scalar_prefetch=0,
            grid=(nbg,),
            in_specs=[
                pl.BlockSpec((G * B * 3, 1024), lambda b: (b, 0)),
                pl.BlockSpec((10 * 96, LANES), lambda b: (0, 0)),
                pl.BlockSpec((K * CL, LANES), lambda b: (0, 0)),
                pl.BlockSpec((K * K * LANES, LANES), lambda b: (0, 0)),
                pl.BlockSpec((K * K * LANES, LANES), lambda b: (0, 0)),
                pl.BlockSpec((1, LANES), lambda b: (0, 0)),
                pl.BlockSpec((1, LANES), lambda b: (0, 0)),
                pl.BlockSpec((1, LANES), lambda b: (0, 0)),
            ],
            out_specs=pl.BlockSpec((G * LANES, 80), lambda b: (b, 0)),
            scratch_shapes=[
                pltpu.VMEM((2 * G * IN1, 2 * CL), jnp.bfloat16),
                pltpu.VMEM((G * IN1, CL), jnp.bfloat16),
                pltpu.VMEM((M1, LANES), jnp.float32),
                pltpu.VMEM((A1R, LANES), jnp.bfloat16),
                pltpu.VMEM((4 * G * P2, LANES), jnp.bfloat16),
                pltpu.VMEM((M2, LANES), jnp.float32),
                pltpu.VMEM((M2, LANES), jnp.float32),
            ],
        ),
        compiler_params=pltpu.CompilerParams(dimension_semantics=("parallel",)),
    )(y, w1p, w14, w2a, w2b, b1p, b2a, b2b)

    wf1r = wf1.reshape(K, K, LANES, LANES)[:, :, :16, :]
    wf1r = wf1r.reshape(K, K, 2, 8, LANES).transpose(3, 2, 0, 1, 4)
    wf1r = jnp.pad(wf1r, ((0, 0), (0, 0), (0, 0), (0, 3), (0, 0)))
    wf1r = wf1r.reshape(640, LANES).astype(jnp.bfloat16)
    wf2b = wf2.astype(jnp.bfloat16)
    wf3b = wf3.astype(jnp.bfloat16)

    tile = FC_TILE if n % FC_TILE == 0 else n
    out = pl.pallas_call(
        functools.partial(_fc_kernel, tile),
        out_shape=jax.ShapeDtypeStruct((n, LANES), jnp.float32),
        grid_spec=pltpu.PrefetchScalarGridSpec(
            num_scalar_prefetch=0,
            grid=(n // tile,),
            in_specs=[
                pl.BlockSpec((8 * tile, 80), lambda b: (b, 0)),
                pl.BlockSpec((640, LANES), lambda b: (0, 0)),
                pl.BlockSpec((1, LANES), lambda b: (0, 0)),
                pl.BlockSpec((LANES, LANES), lambda b: (0, 0)),
                pl.BlockSpec((1, LANES), lambda b: (0, 0)),
                pl.BlockSpec((LANES, LANES), lambda b: (0, 0)),
                pl.BlockSpec((1, LANES), lambda b: (0, 0)),
            ],
            out_specs=pl.BlockSpec((tile, LANES), lambda b: (b, 0)),
            scratch_shapes=[pltpu.VMEM((tile, 640), jnp.bfloat16)],
        ),
        compiler_params=pltpu.CompilerParams(dimension_semantics=("parallel",)),
    )(feats, wf1r, bf1, wf2b, bf2, wf3b, bf3)

    return out[:n, :10]
